# Initial kernel scaffold; baseline (speedup 1.0000x reference)
import functools

import jax
import jax.numpy as jnp
from jax import lax
from jax.experimental import pallas as pl
from jax.experimental.pallas import tpu as pltpu
from jax.experimental.pallas import tpu_sc as plsc

N = 10000
E = 320000
C = 128
NP = 10240
NC = 2
NS = 16
NW = NC * NS
EPW = E // NW
CH = 80
NCHUNK = EPW // CH
RPS = NP // NS

_mesh = plsc.VectorSubcoreMesh(core_axis_name="c", subcore_axis_name="s")

ROWBLK = 1280
GRID = NP // ROWBLK


@functools.partial(
    pl.kernel,
    out_type=jax.ShapeDtypeStruct((NW, NP), jnp.float32),
    mesh=_mesh,
    scratch_types=[
        pltpu.VMEM((EPW,), jnp.int32),
        pltpu.VMEM((NP,), jnp.float32),
    ],
)
def _deg_kernel(dst_hbm, out_hbm, idx_v, cnt_v):
    cid = lax.axis_index("c")
    sid = lax.axis_index("s")
    wid = sid * NC + cid
    zeros = jnp.zeros((16,), jnp.float32)
    ones = jnp.ones((16,), jnp.float32)

    def _zero(i, carry):
        cnt_v[pl.ds(i * 16, 16)] = zeros
        return carry

    lax.fori_loop(0, NP // 16, _zero, 0)
    pltpu.sync_copy(dst_hbm.at[pl.ds(wid * EPW, EPW)], idx_v)

    def _count(j, carry):
        iv = idx_v[pl.ds(j * 16, 16)]
        plsc.addupdate_scatter(cnt_v, [iv], ones)
        return carry

    lax.fori_loop(0, EPW // 16, _count, 0)
    pltpu.sync_copy(cnt_v, out_hbm.at[wid])


@functools.partial(
    pl.kernel,
    out_type=jax.ShapeDtypeStruct((NC, NP, C), jnp.float32),
    mesh=_mesh,
    scratch_types=[
        pltpu.VMEM((NCHUNK, CH), jnp.int32),
        pltpu.VMEM((NCHUNK, CH), jnp.int32),
        pltpu.VMEM((CH, C), jnp.float32),
        pltpu.VMEM_SHARED((NP, C), jnp.float32),
        pltpu.SemaphoreType.DMA,
    ],
)
def _agg_kernel(h_hbm, src_hbm, dst_hbm, out_hbm, src_v, dst_v, rows_v, acc, sem):
    cid = lax.axis_index("c")
    sid = lax.axis_index("s")
    wid = sid * NC + cid
    zeros = jnp.zeros((16,), jnp.float32)

    def _zero(i, carry):
        r = i // (C // 16)
        c = (i % (C // 16)) * 16
        rows_v[r, pl.ds(c, 16)] = zeros
        return carry

    lax.fori_loop(0, CH * C // 16, _zero, 0)

    def _blast(k, carry):
        pltpu.sync_copy(rows_v, acc.at[pl.ds(sid * RPS + k * CH, CH)])
        return carry

    lax.fori_loop(0, RPS // CH, _blast, 0)
    plsc.subcore_barrier()

    pltpu.sync_copy(src_hbm.at[wid], src_v)
    pltpu.sync_copy(dst_hbm.at[wid], dst_v)

    def _chunk(j, carry):
        pltpu.async_copy(h_hbm.at[src_v.at[j]], rows_v, sem).wait()
        pltpu.sync_copy(rows_v, acc.at[dst_v.at[j]], add=True)
        return carry

    lax.fori_loop(0, NCHUNK, _chunk, 0)
    plsc.subcore_barrier()

    pltpu.sync_copy(
        acc.at[pl.ds(sid * RPS, RPS)], out_hbm.at[cid, pl.ds(sid * RPS, RPS)]
    )


def _norm_from(degp):
    deg = jnp.sum(degp, axis=0)
    return lax.rsqrt(jnp.clip(deg, 1.0, None))


def _tcA_body(degp_ref, x_ref, wres_ref, bres_ref, h0_ref, res_ref):
    norm = _norm_from(degp_ref[...])
    x = x_ref[...]
    h0_ref[...] = x * norm[:, None]
    res_ref[...] = (
        lax.dot_general(
            x, wres_ref[...], (((1,), (1,)), ((), ())),
            preferred_element_type=jnp.float32,
        )
        + bres_ref[...]
    )


def _tcB_body(degp_ref, p_ref, w0_ref, b0_ref, res_ref, h_ref, h1s_ref):
    norm = _norm_from(degp_ref[...])
    agg = (p_ref[0] + p_ref[1]) * norm[:, None]
    conv = (
        lax.dot_general(
            agg, w0_ref[...], (((1,), (1,)), ((), ())),
            preferred_element_type=jnp.float32,
        )
        + b0_ref[...]
    )
    h = jnp.maximum(conv, 0.0) + res_ref[...]
    h_ref[...] = h
    h1s_ref[...] = h * norm[:, None]


def _tcC_body(degp_ref, p_ref, w1_ref, b1_ref, h_ref, out_ref):
    norm = _norm_from(degp_ref[...])
    agg = (p_ref[0] + p_ref[1]) * norm[:, None]
    conv = (
        lax.dot_general(
            agg, w1_ref[...], (((1,), (1,)), ((), ())),
            preferred_element_type=jnp.float32,
        )
        + b1_ref[...]
    )
    out_ref[...] = jnp.maximum(conv, 0.0) + h_ref[...]


_degp_spec = pl.BlockSpec((NW, ROWBLK), lambda i: (0, i))
_row_spec = pl.BlockSpec((ROWBLK, C), lambda i: (i, 0))
_p_spec = pl.BlockSpec((NC, ROWBLK, C), lambda i: (0, i, 0))
_w_spec = pl.BlockSpec((C, C), lambda i: (0, 0))
_b_spec = pl.BlockSpec((C,), lambda i: (0,))

_tcA = pl.pallas_call(
    _tcA_body,
    grid=(GRID,),
    in_specs=[_degp_spec, _row_spec, _w_spec, _b_spec],
    out_specs=[_row_spec, _row_spec],
    out_shape=[
        jax.ShapeDtypeStruct((NP, C), jnp.float32),
        jax.ShapeDtypeStruct((NP, C), jnp.float32),
    ],
)

_tcB = pl.pallas_call(
    _tcB_body,
    grid=(GRID,),
    in_specs=[_degp_spec, _p_spec, _w_spec, _b_spec, _row_spec],
    out_specs=[_row_spec, _row_spec],
    out_shape=[
        jax.ShapeDtypeStruct((NP, C), jnp.float32),
        jax.ShapeDtypeStruct((NP, C), jnp.float32),
    ],
)

_tcC = pl.pallas_call(
    _tcC_body,
    grid=(GRID,),
    in_specs=[_degp_spec, _p_spec, _w_spec, _b_spec, _row_spec],
    out_specs=_row_spec,
    out_shape=jax.ShapeDtypeStruct((NP, C), jnp.float32),
)


def kernel(features, src_id, dst_id, W0, b0, W1, b1, Wres, bres):
    src = jnp.reshape(src_id.astype(jnp.int32), (NW, NCHUNK, CH))
    dst = jnp.reshape(dst_id.astype(jnp.int32), (NW, NCHUNK, CH))
    dst_flat = dst_id.astype(jnp.int32)
    x = jnp.pad(features, ((0, NP - N), (0, 0)))

    degp = _deg_kernel(dst_flat)
    h0, res = _tcA(degp, x, Wres, bres)
    p0 = _agg_kernel(h0, src, dst)
    h, h1s = _tcB(degp, p0, W0, b0, res)
    p1 = _agg_kernel(h1s, src, dst)
    out = _tcC(degp, p1, W1, b1, h)
    return out[:N]


# speedup vs baseline: 4.2229x; 4.2229x over previous
import functools

import jax
import jax.numpy as jnp
from jax import lax
from jax.experimental import pallas as pl
from jax.experimental.pallas import tpu as pltpu
from jax.experimental.pallas import tpu_sc as plsc

N = 10000
E = 320000
C = 128
NP = 10240
NC = 2
NS = 16
NW = NC * NS
EPW = E // NW
CH = 80
NCHUNK = EPW // CH
RPS = NP // NS

_mesh = plsc.VectorSubcoreMesh(core_axis_name="c", subcore_axis_name="s")

ROWBLK = 1280
GRID = NP // ROWBLK


@functools.partial(
    pl.kernel,
    out_type=jax.ShapeDtypeStruct((NC, NP, C), jnp.float32),
    mesh=_mesh,
    scratch_types=[
        pltpu.VMEM((CH,), jnp.int32),
        pltpu.VMEM((CH, C), jnp.float32),
        pltpu.VMEM_SHARED((NP, C), jnp.float32),
    ],
)
def _deg_kernel(dst_hbm, out_hbm, didx, ones_v, acc):
    cid = lax.axis_index("c")
    sid = lax.axis_index("s")
    wid = sid * NC + cid
    zeros = jnp.zeros((16,), jnp.float32)

    def _zero(i, carry):
        r = i // (C // 16)
        c = (i % (C // 16)) * 16
        ones_v[r, pl.ds(c, 16)] = zeros
        return carry

    lax.fori_loop(0, CH * C // 16, _zero, 0)

    def _blast(k, carry):
        pltpu.sync_copy(ones_v, acc.at[pl.ds(sid * RPS + k * CH, CH)])
        return carry

    lax.fori_loop(0, RPS // CH, _blast, 0)

    ones = jnp.ones((16,), jnp.float32)

    def _fill(i, carry):
        r = i // (C // 16)
        c = (i % (C // 16)) * 16
        ones_v[r, pl.ds(c, 16)] = ones
        return carry

    lax.fori_loop(0, CH * C // 16, _fill, 0)
    plsc.subcore_barrier()

    base = wid * EPW

    def _chunk(j, carry):
        pltpu.sync_copy(dst_hbm.at[pl.ds(base + j * CH, CH)], didx)
        pltpu.sync_copy(ones_v, acc.at[didx], add=True)
        return carry

    lax.fori_loop(0, NCHUNK, _chunk, 0)
    plsc.subcore_barrier()

    def _out(k, carry):
        r = sid * RPS + k * CH
        pltpu.sync_copy(acc.at[pl.ds(r, CH)], ones_v)
        pltpu.sync_copy(ones_v, out_hbm.at[cid, pl.ds(r, CH)])
        return carry

    lax.fori_loop(0, RPS // CH, _out, 0)


@functools.partial(
    pl.kernel,
    out_type=jax.ShapeDtypeStruct((NC, NP, C), jnp.float32),
    mesh=_mesh,
    scratch_types=[
        pltpu.VMEM((CH,), jnp.int32),
        pltpu.VMEM((CH,), jnp.int32),
        pltpu.VMEM((CH, C), jnp.float32),
        pltpu.VMEM_SHARED((NP, C), jnp.float32),
        pltpu.SemaphoreType.DMA,
    ],
)
def _agg_kernel(h_hbm, src_hbm, dst_hbm, out_hbm, sidx, didx, rows_v, acc, sem):
    cid = lax.axis_index("c")
    sid = lax.axis_index("s")
    wid = sid * NC + cid
    zeros = jnp.zeros((16,), jnp.float32)

    def _zero(i, carry):
        r = i // (C // 16)
        c = (i % (C // 16)) * 16
        rows_v[r, pl.ds(c, 16)] = zeros
        return carry

    lax.fori_loop(0, CH * C // 16, _zero, 0)

    def _blast(k, carry):
        pltpu.sync_copy(rows_v, acc.at[pl.ds(sid * RPS + k * CH, CH)])
        return carry

    lax.fori_loop(0, RPS // CH, _blast, 0)
    plsc.subcore_barrier()

    base = wid * EPW

    def _chunk(j, carry):
        off = base + j * CH
        pltpu.sync_copy(src_hbm.at[pl.ds(off, CH)], sidx)
        pltpu.sync_copy(dst_hbm.at[pl.ds(off, CH)], didx)
        pltpu.async_copy(h_hbm.at[sidx], rows_v, sem).wait()
        pltpu.sync_copy(rows_v, acc.at[didx], add=True)
        return carry

    lax.fori_loop(0, NCHUNK, _chunk, 0)
    plsc.subcore_barrier()

    def _out(k, carry):
        r = sid * RPS + k * CH
        pltpu.sync_copy(acc.at[pl.ds(r, CH)], rows_v)
        pltpu.sync_copy(rows_v, out_hbm.at[cid, pl.ds(r, CH)])
        return carry

    lax.fori_loop(0, RPS // CH, _out, 0)


def _norm_from(degp):
    deg = degp[0] + degp[1]
    return lax.rsqrt(jnp.clip(deg, 1.0, None))


def _tcA_body(degp_ref, x_ref, wres_ref, bres_ref, h0_ref, res_ref):
    normc = _norm_from(degp_ref[...])
    x = x_ref[...]
    h0_ref[...] = x * normc
    res_ref[...] = (
        lax.dot_general(
            x, wres_ref[...], (((1,), (1,)), ((), ())),
            preferred_element_type=jnp.float32,
        )
        + bres_ref[...]
    )


def _tcB_body(degp_ref, p_ref, w0_ref, b0_ref, res_ref, h_ref, h1s_ref):
    normc = _norm_from(degp_ref[...])
    agg = (p_ref[0] + p_ref[1]) * normc
    conv = (
        lax.dot_general(
            agg, w0_ref[...], (((1,), (1,)), ((), ())),
            preferred_element_type=jnp.float32,
        )
        + b0_ref[...]
    )
    h = jnp.maximum(conv, 0.0) + res_ref[...]
    h_ref[...] = h
    h1s_ref[...] = h * normc


def _tcC_body(degp_ref, p_ref, w1_ref, b1_ref, h_ref, out_ref):
    normc = _norm_from(degp_ref[...])
    agg = (p_ref[0] + p_ref[1]) * normc
    conv = (
        lax.dot_general(
            agg, w1_ref[...], (((1,), (1,)), ((), ())),
            preferred_element_type=jnp.float32,
        )
        + b1_ref[...]
    )
    out_ref[...] = jnp.maximum(conv, 0.0) + h_ref[...]


_row_spec = pl.BlockSpec((ROWBLK, C), lambda i: (i, 0))
_p_spec = pl.BlockSpec((NC, ROWBLK, C), lambda i: (0, i, 0))
_w_spec = pl.BlockSpec((C, C), lambda i: (0, 0))
_b_spec = pl.BlockSpec((C,), lambda i: (0,))

_tcA = pl.pallas_call(
    _tcA_body,
    grid=(GRID,),
    in_specs=[_p_spec, _row_spec, _w_spec, _b_spec],
    out_specs=[_row_spec, _row_spec],
    out_shape=[
        jax.ShapeDtypeStruct((NP, C), jnp.float32),
        jax.ShapeDtypeStruct((NP, C), jnp.float32),
    ],
)

_tcB = pl.pallas_call(
    _tcB_body,
    grid=(GRID,),
    in_specs=[_p_spec, _p_spec, _w_spec, _b_spec, _row_spec],
    out_specs=[_row_spec, _row_spec],
    out_shape=[
        jax.ShapeDtypeStruct((NP, C), jnp.float32),
        jax.ShapeDtypeStruct((NP, C), jnp.float32),
    ],
)

_tcC = pl.pallas_call(
    _tcC_body,
    grid=(GRID,),
    in_specs=[_p_spec, _p_spec, _w_spec, _b_spec, _row_spec],
    out_specs=_row_spec,
    out_shape=jax.ShapeDtypeStruct((NP, C), jnp.float32),
)


def kernel(features, src_id, dst_id, W0, b0, W1, b1, Wres, bres):
    src = src_id.astype(jnp.int32)
    dst = dst_id.astype(jnp.int32)
    x = jnp.pad(features, ((0, NP - N), (0, 0)))

    degp = _deg_kernel(dst)
    h0, res = _tcA(degp, x, Wres, bres)
    p0 = _agg_kernel(h0, src, dst)
    h, h1s = _tcB(degp, p0, W0, b0, res)
    p1 = _agg_kernel(h1s, src, dst)
    out = _tcC(degp, p1, W1, b1, h)
    return out[:N]


# speedup vs baseline: 9.2075x; 2.1804x over previous
import functools

import jax
import jax.numpy as jnp
from jax import lax
from jax.experimental import pallas as pl
from jax.experimental.pallas import tpu as pltpu
from jax.experimental.pallas import tpu_sc as plsc

N = 10000
E = 320000
C = 128
NP = 10240
NC = 2
NS = 16
NW = NC * NS
EPW = E // NW
CH = 80
NCHUNK = EPW // CH
RPS = NP // NS

_mesh = plsc.VectorSubcoreMesh(core_axis_name="c", subcore_axis_name="s")

ROWBLK = 1280
GRID = NP // ROWBLK


NPAIR = NCHUNK // 2


@functools.partial(
    pl.kernel,
    out_type=jax.ShapeDtypeStruct((NC, NP, C), jnp.float32),
    mesh=_mesh,
    scratch_types=[
        pltpu.VMEM((CH,), jnp.int32),
        pltpu.VMEM((CH,), jnp.int32),
        pltpu.VMEM((CH, C), jnp.float32),
        pltpu.VMEM_SHARED((NP, C), jnp.float32),
        pltpu.SemaphoreType.DMA,
        pltpu.SemaphoreType.DMA,
    ],
)
def _deg_kernel(dst_hbm, out_hbm, didx0, didx1, ones_v, acc, isem0, isem1):
    cid = lax.axis_index("c")
    sid = lax.axis_index("s")
    wid = sid * NC + cid
    zeros = jnp.zeros((16,), jnp.float32)

    def _zero(i, carry):
        r = i // (C // 16)
        c = (i % (C // 16)) * 16
        ones_v[r, pl.ds(c, 16)] = zeros
        return carry

    lax.fori_loop(0, CH * C // 16, _zero, 0)

    def _blast(k, carry):
        pltpu.sync_copy(ones_v, acc.at[pl.ds(sid * RPS + k * CH, CH)])
        return carry

    lax.fori_loop(0, RPS // CH, _blast, 0)

    ones = jnp.ones((16,), jnp.float32)

    def _fill(i, carry):
        r = i // (C // 16)
        c = (i % (C // 16)) * 16
        ones_v[r, pl.ds(c, 16)] = ones
        return carry

    lax.fori_loop(0, CH * C // 16, _fill, 0)
    plsc.subcore_barrier()

    base = wid * EPW

    def _iload(j, buf, sem):
        pltpu.async_copy(dst_hbm.at[pl.ds(base + j * CH, CH)], buf, sem)

    def _iwait(buf, sem):
        pltpu.make_async_copy(dst_hbm.at[pl.ds(base, CH)], buf, sem).wait()

    _iload(0, didx0, isem0)
    _iload(1, didx1, isem1)

    def _pair(t, carry):
        j = 2 * t
        _iwait(didx0, isem0)
        pltpu.sync_copy(ones_v, acc.at[didx0], add=True)
        _iload(j + 2, didx0, isem0)
        _iwait(didx1, isem1)
        pltpu.sync_copy(ones_v, acc.at[didx1], add=True)

        @pl.when(j + 3 < NCHUNK)
        def _():
            _iload(j + 3, didx1, isem1)

        return carry

    lax.fori_loop(0, NPAIR, _pair, 0)
    _iwait(didx0, isem0)
    pltpu.sync_copy(ones_v, acc.at[didx0], add=True)
    plsc.subcore_barrier()

    def _out(k, carry):
        r = sid * RPS + k * CH
        pltpu.sync_copy(acc.at[pl.ds(r, CH)], ones_v)
        pltpu.sync_copy(ones_v, out_hbm.at[cid, pl.ds(r, CH)])
        return carry

    lax.fori_loop(0, RPS // CH, _out, 0)


NTRIPLE = (NCHUNK - 2) // 3


@functools.partial(
    pl.kernel,
    out_type=jax.ShapeDtypeStruct((NC, NP, C), jnp.float32),
    mesh=_mesh,
    scratch_types=[
        pltpu.VMEM((CH,), jnp.int32),
        pltpu.VMEM((CH,), jnp.int32),
        pltpu.VMEM((CH,), jnp.int32),
        pltpu.VMEM((CH,), jnp.int32),
        pltpu.VMEM((CH,), jnp.int32),
        pltpu.VMEM((CH,), jnp.int32),
        pltpu.VMEM((CH, C), jnp.float32),
        pltpu.VMEM((CH, C), jnp.float32),
        pltpu.VMEM((CH, C), jnp.float32),
        pltpu.VMEM_SHARED((NP, C), jnp.float32),
        pltpu.SemaphoreType.DMA,
        pltpu.SemaphoreType.DMA,
        pltpu.SemaphoreType.DMA,
        pltpu.SemaphoreType.DMA,
        pltpu.SemaphoreType.DMA,
        pltpu.SemaphoreType.DMA,
    ],
)
def _agg_kernel(h_hbm, src_hbm, dst_hbm, out_hbm,
                sidx0, didx0, sidx1, didx1, sidx2, didx2,
                rows0, rows1, rows2, acc,
                isem0, isem1, isem2, gsem0, gsem1, gsem2):
    cid = lax.axis_index("c")
    sid = lax.axis_index("s")
    wid = sid * NC + cid
    zeros = jnp.zeros((16,), jnp.float32)

    def _zero(i, carry):
        r = i // (C // 16)
        c = (i % (C // 16)) * 16
        rows0[r, pl.ds(c, 16)] = zeros
        return carry

    lax.fori_loop(0, CH * C // 16, _zero, 0)

    def _blast(k, carry):
        pltpu.sync_copy(rows0, acc.at[pl.ds(sid * RPS + k * CH, CH)])
        return carry

    lax.fori_loop(0, RPS // CH, _blast, 0)
    plsc.subcore_barrier()

    base = wid * EPW

    def _iload(j, sbuf, dbuf, sem):
        pltpu.async_copy(src_hbm.at[pl.ds(base + j * CH, CH)], sbuf, sem)
        pltpu.async_copy(dst_hbm.at[pl.ds(base + j * CH, CH)], dbuf, sem)

    def _iwait(sbuf, dbuf, sem):
        pltpu.make_async_copy(src_hbm.at[pl.ds(base, CH)], sbuf, sem).wait()
        pltpu.make_async_copy(dst_hbm.at[pl.ds(base, CH)], dbuf, sem).wait()

    def _gwait(rows, sem):
        pltpu.make_async_copy(h_hbm.at[pl.ds(0, CH)], rows, sem).wait()

    pltpu.sync_copy(src_hbm.at[pl.ds(base, CH)], sidx0)
    pltpu.sync_copy(dst_hbm.at[pl.ds(base, CH)], didx0)
    pltpu.async_copy(h_hbm.at[sidx0], rows0, gsem0)
    _iload(1, sidx1, didx1, isem1)
    _iload(2, sidx2, didx2, isem2)

    def _triple(t, carry):
        j = 3 * t
        _iwait(sidx1, didx1, isem1)
        pltpu.async_copy(h_hbm.at[sidx1], rows1, gsem1)
        _gwait(rows0, gsem0)
        pltpu.sync_copy(rows0, acc.at[didx0], add=True)
        _iload(j + 3, sidx0, didx0, isem0)
        _iwait(sidx2, didx2, isem2)
        pltpu.async_copy(h_hbm.at[sidx2], rows2, gsem2)
        _gwait(rows1, gsem1)
        pltpu.sync_copy(rows1, acc.at[didx1], add=True)
        _iload(j + 4, sidx1, didx1, isem1)
        _iwait(sidx0, didx0, isem0)
        pltpu.async_copy(h_hbm.at[sidx0], rows0, gsem0)
        _gwait(rows2, gsem2)
        pltpu.sync_copy(rows2, acc.at[didx2], add=True)

        @pl.when(j + 5 < NCHUNK)
        def _():
            _iload(j + 5, sidx2, didx2, isem2)

        return carry

    lax.fori_loop(0, NTRIPLE, _triple, 0)
    _iwait(sidx1, didx1, isem1)
    pltpu.async_copy(h_hbm.at[sidx1], rows1, gsem1)
    _gwait(rows0, gsem0)
    pltpu.sync_copy(rows0, acc.at[didx0], add=True)
    _gwait(rows1, gsem1)
    pltpu.sync_copy(rows1, acc.at[didx1], add=True)
    plsc.subcore_barrier()

    def _out(k, carry):
        r = sid * RPS + k * CH
        pltpu.sync_copy(acc.at[pl.ds(r, CH)], rows0)
        pltpu.sync_copy(rows0, out_hbm.at[cid, pl.ds(r, CH)])
        return carry

    lax.fori_loop(0, RPS // CH, _out, 0)


def _norm_from(degp):
    deg = degp[0] + degp[1]
    return lax.rsqrt(jnp.clip(deg, 1.0, None))


def _tcA_body(degp_ref, x_ref, wres_ref, bres_ref, h0_ref, res_ref):
    normc = _norm_from(degp_ref[...])
    x = x_ref[...]
    h0_ref[...] = x * normc
    res_ref[...] = (
        lax.dot_general(
            x, wres_ref[...], (((1,), (1,)), ((), ())),
            preferred_element_type=jnp.float32,
        )
        + bres_ref[...]
    )


def _tcB_body(degp_ref, p_ref, w0_ref, b0_ref, res_ref, h_ref, h1s_ref):
    normc = _norm_from(degp_ref[...])
    agg = (p_ref[0] + p_ref[1]) * normc
    conv = (
        lax.dot_general(
            agg, w0_ref[...], (((1,), (1,)), ((), ())),
            preferred_element_type=jnp.float32,
        )
        + b0_ref[...]
    )
    h = jnp.maximum(conv, 0.0) + res_ref[...]
    h_ref[...] = h
    h1s_ref[...] = h * normc


def _tcC_body(degp_ref, p_ref, w1_ref, b1_ref, h_ref, out_ref):
    normc = _norm_from(degp_ref[...])
    agg = (p_ref[0] + p_ref[1]) * normc
    conv = (
        lax.dot_general(
            agg, w1_ref[...], (((1,), (1,)), ((), ())),
            preferred_element_type=jnp.float32,
        )
        + b1_ref[...]
    )
    out_ref[...] = jnp.maximum(conv, 0.0) + h_ref[...]


_row_spec = pl.BlockSpec((ROWBLK, C), lambda i: (i, 0))
_p_spec = pl.BlockSpec((NC, ROWBLK, C), lambda i: (0, i, 0))
_w_spec = pl.BlockSpec((C, C), lambda i: (0, 0))
_b_spec = pl.BlockSpec((C,), lambda i: (0,))

_tcA = pl.pallas_call(
    _tcA_body,
    grid=(GRID,),
    in_specs=[_p_spec, _row_spec, _w_spec, _b_spec],
    out_specs=[_row_spec, _row_spec],
    out_shape=[
        jax.ShapeDtypeStruct((NP, C), jnp.float32),
        jax.ShapeDtypeStruct((NP, C), jnp.float32),
    ],
)

_tcB = pl.pallas_call(
    _tcB_body,
    grid=(GRID,),
    in_specs=[_p_spec, _p_spec, _w_spec, _b_spec, _row_spec],
    out_specs=[_row_spec, _row_spec],
    out_shape=[
        jax.ShapeDtypeStruct((NP, C), jnp.float32),
        jax.ShapeDtypeStruct((NP, C), jnp.float32),
    ],
)

_tcC = pl.pallas_call(
    _tcC_body,
    grid=(GRID,),
    in_specs=[_p_spec, _p_spec, _w_spec, _b_spec, _row_spec],
    out_specs=_row_spec,
    out_shape=jax.ShapeDtypeStruct((NP, C), jnp.float32),
)


def kernel(features, src_id, dst_id, W0, b0, W1, b1, Wres, bres):
    src = src_id.astype(jnp.int32)
    dst = dst_id.astype(jnp.int32)
    x = jnp.pad(features, ((0, NP - N), (0, 0)))

    degp = _deg_kernel(dst)
    h0, res = _tcA(degp, x, Wres, bres)
    p0 = _agg_kernel(h0, src, dst)
    h, h1s = _tcB(degp, p0, W0, b0, res)
    p1 = _agg_kernel(h1s, src, dst)
    out = _tcC(degp, p1, W1, b1, h)
    return out[:N]
